# baseline (speedup 1.0000x reference)
import functools

import jax
import jax.numpy as jnp
from jax import lax
from jax.experimental import pallas as pl
from jax.experimental.pallas import tpu as pltpu
from jax.experimental.pallas import tpu_sc as plsc

NC, NS, L = 2, 16, 16
NW = NC * NS

K = 64
V = 513
VA = 512
D = 768
R = 512
RPW = R // NW
IPW = RPW * K
NV = D // L

_mesh = plsc.VectorSubcoreMesh(core_axis_name="c", subcore_axis_name="s")


@functools.partial(
    pl.kernel,
    out_type=jax.ShapeDtypeStruct((R, D), jnp.float32),
    mesh=_mesh,
    scratch_types=[
        pltpu.VMEM((IPW,), jnp.int32),
        pltpu.VMEM((IPW,), jnp.int32),
        pltpu.VMEM((RPW, D), jnp.float32),
        pltpu.VMEM((RPW, D), jnp.float32),
        pltpu.VMEM((RPW, D), jnp.float32),
        pltpu.VMEM_SHARED((K, D), jnp.float32),
        pltpu.VMEM_SHARED((VA, D), jnp.float32),
        pltpu.VMEM_SHARED((VA, D), jnp.float32),
        pltpu.SemaphoreType.DMA,
        pltpu.SemaphoreType.DMA,
        pltpu.SemaphoreType.DMA,
        pltpu.SemaphoreType.DMA,
    ],
    compiler_params=pltpu.CompilerParams(
        use_tc_tiling_on_sc=True, needs_layout_passes=False),
)
def _emb_sum(ids_hbm, table_hbm, last_hbm, out_hbm, idx_v, idxt_v, rows_0,
             rows_1, acc_v, last_sh, sh_0, sh_1, ssem_0, ssem_1, esem_0,
             esem_1):
    cid = lax.axis_index("c")
    sid = lax.axis_index("s")
    wid = sid * NC + cid
    base = wid * IPW
    is_leader = sid == 0
    pltpu.sync_copy(ids_hbm.at[pl.ds(base, IPW)], idx_v)

    @pl.when(is_leader)
    def _():
        pltpu.sync_copy(last_hbm, last_sh)

    lane64 = lax.iota(jnp.int32, L) * K

    def transpose(i, _):
        vec = plsc.load_gather(idx_v, [lane64 + i])
        idxt_v[pl.ds(i * L, L)] = vec
        return 0

    lax.fori_loop(0, K, transpose, 0)

    zero = jnp.zeros((L,), jnp.float32)

    def clear(c, _):
        for j in range(RPW):
            acc_v[j, pl.ds(c * L, L)] = zero
        return 0

    lax.fori_loop(0, NV, clear, 0)

    def stage(i, sh, ssem):
        return pltpu.make_async_copy(table_hbm.at[i, pl.ds(0, VA)], sh,
                                     ssem)

    def extract(i, sh, buf, esem):
        ids_vec = idxt_v[pl.ds(i * L, L)]
        clamped = jnp.minimum(ids_vec, VA - 1)
        for j in range(RPW):
            pltpu.make_async_copy(sh.at[clamped[j]], buf.at[j],
                                  esem).start()
        for j in range(RPW):
            pltpu.make_async_copy(sh.at[0], buf.at[j], esem).wait()
        for j in range(RPW):
            @pl.when(ids_vec[j] == VA)
            def _():
                pltpu.sync_copy(last_sh.at[i], buf.at[j])

    def accum(buf):
        def body(c, _):
            for j in range(RPW):
                plsc.addupdate(acc_v.at[j, pl.ds(c * L, L)],
                               buf[j, pl.ds(c * L, L)])
            return 0

        lax.fori_loop(0, NV, body, 0)

    @pl.when(is_leader)
    def _():
        stage(0, sh_0, ssem_0).start()
        stage(1, sh_1, ssem_1).start()

    shs = (sh_0, sh_1)
    ssems = (ssem_0, ssem_1)
    rows = (rows_0, rows_1)
    esems = (esem_0, esem_1)

    def pair(h, _):
        i0 = 2 * h
        for b in range(2):
            i = i0 + b

            @pl.when(is_leader)
            def _():
                stage(i, shs[b], ssems[b]).wait()

            plsc.subcore_barrier()
            extract(i, shs[b], rows[b], esems[b])
            plsc.subcore_barrier()

            @pl.when(is_leader & (i + 2 < K))
            def _():
                stage(i + 2, shs[b], ssems[b]).start()

            accum(rows[b])
        return 0

    lax.fori_loop(0, K // 2, pair, 0)
    pltpu.sync_copy(acc_v, out_hbm.at[pl.ds(wid * RPW, RPW)])


def kernel(input_ids, tables):
    b, seq = input_ids.shape
    ids = input_ids.astype(jnp.int32).reshape(-1)
    last_rows = tables[:, V - 1, :]
    out = _emb_sum(ids, tables, last_rows)
    return out.reshape(b, seq // K, D)


# speedup vs baseline: 1.0004x; 1.0004x over previous
import functools

import jax
import jax.numpy as jnp
from jax import lax
from jax.experimental import pallas as pl
from jax.experimental.pallas import tpu as pltpu
from jax.experimental.pallas import tpu_sc as plsc

NC, NS, L = 2, 16, 16
NW = NC * NS

K = 64
V = 513
VA = 512
D = 768
R = 512
RPW = R // NW
IPW = RPW * K
NV = D // L

_mesh = plsc.VectorSubcoreMesh(core_axis_name="c", subcore_axis_name="s")


@functools.partial(
    pl.kernel,
    out_type=jax.ShapeDtypeStruct((R, D), jnp.float32),
    mesh=_mesh,
    scratch_types=[
        pltpu.VMEM((IPW,), jnp.int32),
        pltpu.VMEM((IPW,), jnp.int32),
        pltpu.VMEM((RPW, D), jnp.float32),
        pltpu.VMEM((RPW, D), jnp.float32),
        pltpu.VMEM((RPW, D), jnp.float32),
        pltpu.VMEM_SHARED((K, D), jnp.float32),
        pltpu.VMEM_SHARED((VA, D), jnp.float32),
        pltpu.VMEM_SHARED((VA, D), jnp.float32),
        pltpu.SemaphoreType.DMA,
        pltpu.SemaphoreType.DMA,
        pltpu.SemaphoreType.DMA,
        pltpu.SemaphoreType.DMA,
    ],
    compiler_params=pltpu.CompilerParams(
        use_tc_tiling_on_sc=True, needs_layout_passes=False),
)
def _emb_sum(ids_hbm, table_hbm, last_hbm, out_hbm, idx_v, idxt_v, rows_0,
             rows_1, acc_v, last_sh, sh_0, sh_1, ssem_0, ssem_1, esem_0,
             esem_1):
    cid = lax.axis_index("c")
    sid = lax.axis_index("s")
    wid = sid * NC + cid
    base = wid * IPW
    is_leader = sid == 0
    pltpu.sync_copy(ids_hbm.at[pl.ds(base, IPW)], idx_v)

    @pl.when(is_leader)
    def _():
        pltpu.sync_copy(last_hbm, last_sh)

    lane64 = lax.iota(jnp.int32, L) * K

    def transpose(i, _):
        vec = plsc.load_gather(idx_v, [lane64 + i])
        idxt_v[pl.ds(i * L, L)] = vec
        return 0

    lax.fori_loop(0, K, transpose, 0)

    zero = jnp.zeros((L,), jnp.float32)

    def clear(c, _):
        for j in range(RPW):
            acc_v[j, pl.ds(c * L, L)] = zero
        return 0

    lax.fori_loop(0, NV, clear, 0)

    SPT = VA // NS

    def stage(i, sh, ssem):
        return pltpu.make_async_copy(
            table_hbm.at[i, pl.ds(sid * SPT, SPT)],
            sh.at[pl.ds(sid * SPT, SPT)], ssem)

    def extract(i, sh, buf, esem):
        ids_vec = idxt_v[pl.ds(i * L, L)]
        clamped = jnp.minimum(ids_vec, VA - 1)
        for j in range(RPW):
            pltpu.make_async_copy(sh.at[clamped[j]], buf.at[j],
                                  esem).start()
        for j in range(RPW):
            pltpu.make_async_copy(sh.at[0], buf.at[j], esem).wait()
        for j in range(RPW):
            @pl.when(ids_vec[j] == VA)
            def _():
                pltpu.sync_copy(last_sh.at[i], buf.at[j])

    def accum(buf):
        def body(c, _):
            for j in range(RPW):
                plsc.addupdate(acc_v.at[j, pl.ds(c * L, L)],
                               buf[j, pl.ds(c * L, L)])
            return 0

        lax.fori_loop(0, NV, body, 0)

    stage(0, sh_0, ssem_0).start()
    stage(1, sh_1, ssem_1).start()

    shs = (sh_0, sh_1)
    ssems = (ssem_0, ssem_1)
    rows = (rows_0, rows_1)
    esems = (esem_0, esem_1)

    def pair(h, _):
        i0 = 2 * h
        for b in range(2):
            i = i0 + b

            stage(i, shs[b], ssems[b]).wait()
            plsc.subcore_barrier()
            extract(i, shs[b], rows[b], esems[b])
            plsc.subcore_barrier()

            @pl.when(i + 2 < K)
            def _():
                stage(i + 2, shs[b], ssems[b]).start()

            accum(rows[b])
        return 0

    lax.fori_loop(0, K // 2, pair, 0)
    pltpu.sync_copy(acc_v, out_hbm.at[pl.ds(wid * RPW, RPW)])


def kernel(input_ids, tables):
    b, seq = input_ids.shape
    ids = input_ids.astype(jnp.int32).reshape(-1)
    last_rows = tables[:, V - 1, :]
    out = _emb_sum(ids, tables, last_rows)
    return out.reshape(b, seq // K, D)


# speedup vs baseline: 1.5622x; 1.5616x over previous
import functools

import jax
import jax.numpy as jnp
from jax import lax
from jax.experimental import pallas as pl
from jax.experimental.pallas import tpu as pltpu
from jax.experimental.pallas import tpu_sc as plsc

NC, NS, L = 2, 16, 16
NW = NC * NS

K = 64
V = 513
D = 768
R = 512
RPW = R // NW
IPW = RPW * K
NV = D // L

_mesh = plsc.VectorSubcoreMesh(core_axis_name="c", subcore_axis_name="s")


@functools.partial(
    pl.kernel,
    out_type=jax.ShapeDtypeStruct((R, D), jnp.float32),
    mesh=_mesh,
    scratch_types=[
        pltpu.VMEM((IPW,), jnp.int32),
        pltpu.VMEM((IPW,), jnp.int32),
        pltpu.VMEM((RPW, D), jnp.float32),
        pltpu.VMEM((RPW, D), jnp.float32),
        pltpu.VMEM((RPW, D), jnp.float32),
        pltpu.VMEM((RPW, D), jnp.float32),
        pltpu.VMEM((RPW, D), jnp.float32),
        pltpu.SemaphoreType.DMA,
        pltpu.SemaphoreType.DMA,
        pltpu.SemaphoreType.DMA,
        pltpu.SemaphoreType.DMA,
    ],
    compiler_params=pltpu.CompilerParams(use_tc_tiling_on_sc=True, needs_layout_passes=False),
)
def _emb_sum(ids_hbm, table_hbm, out_hbm, idx_v, idxt_v, rows_0, rows_1,
             rows_2, rows_3, acc_v, sem_0, sem_1, sem_2, sem_3):
    wid = lax.axis_index("s") * NC + lax.axis_index("c")
    base = wid * IPW
    pltpu.sync_copy(ids_hbm.at[pl.ds(base, IPW)], idx_v)

    lane64 = lax.iota(jnp.int32, L) * K

    def transpose(i, _):
        vec = plsc.load_gather(idx_v, [lane64 + i])
        idxt_v[pl.ds(i * L, L)] = vec
        return 0

    lax.fori_loop(0, K, transpose, 0)

    zero = jnp.zeros((L,), jnp.float32)

    def clear(c, _):
        for j in range(RPW):
            acc_v[j, pl.ds(c * L, L)] = zero
        return 0

    lax.fori_loop(0, NV, clear, 0)

    def gather(i, buf, sem):
        return pltpu.make_async_copy(
            table_hbm.at[i].at[idxt_v.at[pl.ds(i * L, L)]], buf, sem)

    def accum2(buf_x, buf_y):
        def body(c, _):
            for j in range(RPW):
                s = buf_x[j, pl.ds(c * L, L)] + buf_y[j, pl.ds(c * L, L)]
                plsc.addupdate(acc_v.at[j, pl.ds(c * L, L)], s)
            return 0

        lax.fori_loop(0, NV, body, 0)

    bufs = (rows_0, rows_1, rows_2, rows_3)
    sems = (sem_0, sem_1, sem_2, sem_3)

    for b in range(4):
        gather(b, bufs[b], sems[b]).start()

    def ring(h, _):
        i0 = 4 * h
        for b in (0, 2):
            i = i0 + b
            gather(i, bufs[b], sems[b]).wait()
            gather(i + 1, bufs[b + 1], sems[b + 1]).wait()
            accum2(bufs[b], bufs[b + 1])

            @pl.when(i + 4 < K)
            def _():
                gather(i + 4, bufs[b], sems[b]).start()

            @pl.when(i + 5 < K)
            def _():
                gather(i + 5, bufs[b + 1], sems[b + 1]).start()
        return 0

    lax.fori_loop(0, K // 4, ring, 0)
    pltpu.sync_copy(acc_v, out_hbm.at[pl.ds(wid * RPW, RPW)])


def kernel(input_ids, tables):
    b, seq = input_ids.shape
    ids = input_ids.astype(jnp.int32).reshape(-1)
    out = _emb_sum(ids, tables)
    return out.reshape(b, seq // K, D)


# speedup vs baseline: 1.6646x; 1.0655x over previous
import functools

import jax
import jax.numpy as jnp
from jax import lax
from jax.experimental import pallas as pl
from jax.experimental.pallas import tpu as pltpu
from jax.experimental.pallas import tpu_sc as plsc

NC, NS, L = 2, 16, 16
NW = NC * NS

K = 64
V = 513
D = 768
R = 512
RPW = R // NW
IPW = RPW * K
NV = D // L

_mesh = plsc.VectorSubcoreMesh(core_axis_name="c", subcore_axis_name="s")


@functools.partial(
    pl.kernel,
    out_type=jax.ShapeDtypeStruct((R, D), jnp.float32),
    mesh=_mesh,
    scratch_types=[
        pltpu.VMEM((IPW,), jnp.int32),
        pltpu.VMEM((IPW,), jnp.int32),
        pltpu.VMEM((8, RPW, D), jnp.float32),
        pltpu.VMEM((RPW, D), jnp.float32),
        pltpu.SemaphoreType.DMA,
        pltpu.SemaphoreType.DMA,
        pltpu.SemaphoreType.DMA,
        pltpu.SemaphoreType.DMA,
        pltpu.SemaphoreType.DMA,
        pltpu.SemaphoreType.DMA,
        pltpu.SemaphoreType.DMA,
        pltpu.SemaphoreType.DMA,
    ],
    compiler_params=pltpu.CompilerParams(use_tc_tiling_on_sc=True, needs_layout_passes=False),
)
def _emb_sum(ids_hbm, table_hbm, out_hbm, idx_v, idxt_v, rows_v, acc_v,
             sem_0, sem_1, sem_2, sem_3, sem_4, sem_5, sem_6, sem_7):
    wid = lax.axis_index("s") * NC + lax.axis_index("c")
    base = wid * IPW
    pltpu.sync_copy(ids_hbm.at[pl.ds(base, IPW)], idx_v)

    lane64 = lax.iota(jnp.int32, L) * K

    def transpose(i, _):
        vec = plsc.load_gather(idx_v, [lane64 + i])
        idxt_v[pl.ds(i * L, L)] = vec
        return 0

    lax.fori_loop(0, K, transpose, 0)

    zero = jnp.zeros((L,), jnp.float32)

    def clear(c, _):
        for j in range(RPW):
            acc_v[j, pl.ds(c * L, L)] = zero
        return 0

    lax.fori_loop(0, NV, clear, 0)

    def gather(i, b, sem):
        return pltpu.make_async_copy(
            table_hbm.at[i].at[idxt_v.at[pl.ds(i * L, L)]], rows_v.at[b],
            sem)

    def accum4(b0, b1, b2, b3):
        def body(c, _):
            for j in range(RPW):
                ds = pl.ds(c * L, L)
                s = ((rows_v[b0, j, ds] + rows_v[b1, j, ds]) +
                     (rows_v[b2, j, ds] + rows_v[b3, j, ds]))
                plsc.addupdate(acc_v.at[j, ds], s)
            return 0

        lax.fori_loop(0, NV, body, 0)

    sems = (sem_0, sem_1, sem_2, sem_3, sem_4, sem_5, sem_6, sem_7)

    for b in range(8):
        gather(b, b, sems[b]).start()

    def ring(h, _):
        i0 = 8 * h
        for b in (0, 4):
            i = i0 + b
            for t in range(4):
                gather(i + t, b + t, sems[b + t]).wait()
            accum4(b, b + 1, b + 2, b + 3)
            for t in range(4):
                @pl.when(i + 8 + t < K)
                def _():
                    gather(i + 8 + t, b + t, sems[b + t]).start()
        return 0

    lax.fori_loop(0, K // 8, ring, 0)
    pltpu.sync_copy(acc_v, out_hbm.at[pl.ds(wid * RPW, RPW)])


def kernel(input_ids, tables):
    b, seq = input_ids.shape
    ids = input_ids.astype(jnp.int32).reshape(-1)
    out = _emb_sum(ids, tables)
    return out.reshape(b, seq // K, D)
